# Initial kernel scaffold; baseline (speedup 1.0000x reference)
import jax
import jax.numpy as jnp
from jax.experimental import pallas as pl


def kernel(inputs, embeddings):
    raise NotImplementedError("write your pallas kernel here")


# speedup vs baseline: 1.1203x; 1.1203x over previous
import functools

import jax
import jax.numpy as jnp
from jax import lax
from jax.experimental import pallas as pl
from jax.experimental.pallas import tpu as pltpu
from jax.experimental.pallas import tpu_sc as plsc

NUM_EMBEDDINGS = 8192
EMBEDDING_DIM = 64
COMMITMENT_COST = 0.25

N_ROWS = 8 * 1024
ROW_BLK = 1024
CODE_BLK = 2048
N_RB = N_ROWS // ROW_BLK
N_CB = NUM_EMBEDDINGS // CODE_BLK


def _argmin_body(x_ref, e_ref, idx_ref, loss_ref, minv_ref, mini_ref,
                 mind_ref, acc_ref):
    i = pl.program_id(0)
    j = pl.program_id(1)

    x = x_ref[...]
    e = e_ref[...]
    xb = x.astype(jnp.bfloat16).astype(jnp.float32)
    mm = lax.dot_general(
        xb, e, (((1,), (1,)), ((), ())),
        preferred_element_type=jnp.float32)
    row_sq = jnp.sum(x * x, axis=1, keepdims=True)
    e_sq = jnp.sum(e * e, axis=1)[None, :]
    d = (row_sq + e_sq) - 2.0 * mm

    bmin = jnp.min(d, axis=1, keepdims=True)
    ids = lax.broadcasted_iota(jnp.int32, (ROW_BLK, CODE_BLK), 1) + j * CODE_BLK
    barg = jnp.min(jnp.where(d == bmin, ids, jnp.int32(2 ** 30)),
                   axis=1, keepdims=True)

    bmin_r = bmin.astype(jnp.bfloat16).astype(jnp.float32)

    @pl.when(j == 0)
    def _():
        minv_ref[...] = bmin_r
        mini_ref[...] = barg
        mind_ref[...] = bmin

    @pl.when(j > 0)
    def _():
        better = bmin < minv_ref[...]
        minv_ref[...] = jnp.where(better, bmin_r, minv_ref[...])
        mini_ref[...] = jnp.where(better, barg, mini_ref[...])
        mind_ref[...] = jnp.where(better, bmin, mind_ref[...])

    @pl.when(j == N_CB - 1)
    def _():
        idx_ref[...] = mini_ref[...]
        part = jnp.sum(mind_ref[...])
        acc = jnp.where(i == 0, 0.0, acc_ref[0, 0]) + part
        acc_ref[0, 0] = acc
        scale = (1.0 + COMMITMENT_COST) / (N_ROWS * EMBEDDING_DIM)
        loss_ref[...] = jnp.full((1, 1), acc * scale, jnp.float32)


_argmin_call = pl.pallas_call(
    _argmin_body,
    grid=(N_RB, N_CB),
    in_specs=[
        pl.BlockSpec((ROW_BLK, EMBEDDING_DIM), lambda i, j: (i, 0)),
        pl.BlockSpec((CODE_BLK, EMBEDDING_DIM), lambda i, j: (j, 0)),
    ],
    out_specs=[
        pl.BlockSpec((ROW_BLK, 1), lambda i, j: (i, 0)),
        pl.BlockSpec((1, 1), lambda i, j: (0, 0)),
    ],
    out_shape=[
        jax.ShapeDtypeStruct((N_ROWS, 1), jnp.int32),
        jax.ShapeDtypeStruct((1, 1), jnp.float32),
    ],
    scratch_shapes=[
        pltpu.VMEM((ROW_BLK, 1), jnp.float32),
        pltpu.VMEM((ROW_BLK, 1), jnp.int32),
        pltpu.VMEM((ROW_BLK, 1), jnp.float32),
        pltpu.SMEM((1, 1), jnp.float32),
    ],
)


def _make_gather():
    nc, ns = 2, 16
    nw = nc * ns
    rows_per_w = N_ROWS // nw
    chunks = rows_per_w // 128

    mesh = plsc.VectorSubcoreMesh(core_axis_name="c", subcore_axis_name="s")

    @functools.partial(
        pl.kernel, mesh=mesh,
        compiler_params=pltpu.CompilerParams(use_tc_tiling_on_sc=False),
        out_type=jax.ShapeDtypeStruct((N_ROWS, EMBEDDING_DIM), jnp.float32),
        scratch_types=[
            pltpu.VMEM((chunks, 128), jnp.int32),
            pltpu.VMEM((rows_per_w, EMBEDDING_DIM), jnp.float32),
            pltpu.SemaphoreType.DMA,
        ],
    )
    def gather_k(table_hbm, idx_hbm, out_hbm, idx_v, rows_v, sem):
        wid = lax.axis_index("s") * nc + lax.axis_index("c")
        pltpu.sync_copy(idx_hbm.at[pl.ds(wid * chunks, chunks)], idx_v)
        cps = [
            pltpu.async_copy(table_hbm.at[idx_v.at[k]],
                             rows_v.at[pl.ds(k * 128, 128)], sem)
            for k in range(chunks)
        ]
        for cp in cps:
            cp.wait()
        pltpu.sync_copy(rows_v, out_hbm.at[pl.ds(wid * rows_per_w, rows_per_w)])

    return gather_k


_gather_cache = []


def _gather_call(embeddings, idx_rows):
    if not _gather_cache:
        _gather_cache.append(_make_gather())
    return _gather_cache[0](embeddings, idx_rows)


def kernel(inputs, embeddings):
    input_shape = inputs.shape
    flat = inputs.reshape(-1, EMBEDDING_DIM)
    idx2d, loss11 = _argmin_call(flat, embeddings)
    idx_rows = idx2d.reshape(-1, 128)
    quantized = _gather_call(embeddings, idx_rows).reshape(input_shape)
    return quantized, loss11[0, 0], idx2d


# speedup vs baseline: 1.1606x; 1.0359x over previous
import functools

import jax
import jax.numpy as jnp
from jax import lax
from jax.experimental import pallas as pl
from jax.experimental.pallas import tpu as pltpu
from jax.experimental.pallas import tpu_sc as plsc

NUM_EMBEDDINGS = 8192
EMBEDDING_DIM = 64
COMMITMENT_COST = 0.25

N_ROWS = 8 * 1024
ROW_BLK = 2048
CODE_BLK = 2048
N_RB = N_ROWS // ROW_BLK
N_CB = NUM_EMBEDDINGS // CODE_BLK


def _argmin_body(x_ref, e_ref, idx_ref, loss_ref, minv_ref, mini_ref,
                 mind_ref, acc_ref):
    i = pl.program_id(0)
    j = pl.program_id(1)

    x = x_ref[...]
    e = e_ref[...]
    xb = x.astype(jnp.bfloat16).astype(jnp.float32)
    mm = lax.dot_general(
        xb, e, (((1,), (1,)), ((), ())),
        preferred_element_type=jnp.float32)
    row_sq = jnp.sum(x * x, axis=1, keepdims=True)
    e_sq = jnp.sum(e * e, axis=1)[None, :]
    d = (row_sq + e_sq) - 2.0 * mm

    bmin = jnp.min(d, axis=1, keepdims=True)
    ids = lax.broadcasted_iota(jnp.int32, (ROW_BLK, CODE_BLK), 1)
    barg = jnp.min(jnp.where(d == bmin, ids, jnp.int32(2 ** 30)),
                   axis=1, keepdims=True) + j * CODE_BLK

    bmin_r = bmin.astype(jnp.bfloat16).astype(jnp.float32)

    @pl.when(j == 0)
    def _():
        minv_ref[...] = bmin_r
        mini_ref[...] = barg
        mind_ref[...] = bmin

    @pl.when(j > 0)
    def _():
        better = bmin < minv_ref[...]
        minv_ref[...] = jnp.where(better, bmin_r, minv_ref[...])
        mini_ref[...] = jnp.where(better, barg, mini_ref[...])
        mind_ref[...] = jnp.where(better, bmin, mind_ref[...])

    @pl.when(j == N_CB - 1)
    def _():
        idx_ref[...] = mini_ref[...]
        part = jnp.sum(mind_ref[...])
        acc = jnp.where(i == 0, 0.0, acc_ref[0, 0]) + part
        acc_ref[0, 0] = acc
        scale = (1.0 + COMMITMENT_COST) / (N_ROWS * EMBEDDING_DIM)
        loss_ref[...] = jnp.full((1, 1), acc * scale, jnp.float32)


_argmin_call = pl.pallas_call(
    _argmin_body,
    grid=(N_RB, N_CB),
    in_specs=[
        pl.BlockSpec((ROW_BLK, EMBEDDING_DIM), lambda i, j: (i, 0)),
        pl.BlockSpec((CODE_BLK, EMBEDDING_DIM), lambda i, j: (j, 0)),
    ],
    out_specs=[
        pl.BlockSpec((ROW_BLK, 1), lambda i, j: (i, 0)),
        pl.BlockSpec((1, 1), lambda i, j: (0, 0)),
    ],
    out_shape=[
        jax.ShapeDtypeStruct((N_ROWS, 1), jnp.int32),
        jax.ShapeDtypeStruct((1, 1), jnp.float32),
    ],
    scratch_shapes=[
        pltpu.VMEM((ROW_BLK, 1), jnp.float32),
        pltpu.VMEM((ROW_BLK, 1), jnp.int32),
        pltpu.VMEM((ROW_BLK, 1), jnp.float32),
        pltpu.SMEM((1, 1), jnp.float32),
    ],
)


def _make_gather():
    nc, ns = 2, 16
    nw = nc * ns
    rows_per_w = N_ROWS // nw
    chunks = rows_per_w // 128

    mesh = plsc.VectorSubcoreMesh(core_axis_name="c", subcore_axis_name="s")

    @functools.partial(
        pl.kernel, mesh=mesh,
        compiler_params=pltpu.CompilerParams(use_tc_tiling_on_sc=False),
        out_type=jax.ShapeDtypeStruct((N_ROWS, EMBEDDING_DIM), jnp.float32),
        scratch_types=[
            pltpu.VMEM((chunks, 128), jnp.int32),
            pltpu.VMEM((rows_per_w, EMBEDDING_DIM), jnp.float32),
            pltpu.SemaphoreType.DMA,
        ],
    )
    def gather_k(table_hbm, idx_hbm, out_hbm, idx_v, rows_v, sem):
        wid = lax.axis_index("s") * nc + lax.axis_index("c")
        pltpu.sync_copy(idx_hbm.at[pl.ds(wid * chunks, chunks)], idx_v)
        cps = [
            pltpu.async_copy(table_hbm.at[idx_v.at[k]],
                             rows_v.at[pl.ds(k * 128, 128)], sem)
            for k in range(chunks)
        ]
        for cp in cps:
            cp.wait()
        pltpu.sync_copy(rows_v, out_hbm.at[pl.ds(wid * rows_per_w, rows_per_w)])

    return gather_k


_gather_cache = []


def _gather_call(embeddings, idx_rows):
    if not _gather_cache:
        _gather_cache.append(_make_gather())
    return _gather_cache[0](embeddings, idx_rows)


def kernel(inputs, embeddings):
    input_shape = inputs.shape
    flat = inputs.reshape(-1, EMBEDDING_DIM)
    idx2d, loss11 = _argmin_call(flat, embeddings)
    idx_rows = idx2d.reshape(-1, 128)
    quantized = _gather_call(embeddings, idx_rows).reshape(input_shape)
    return quantized, loss11[0, 0], idx2d


# speedup vs baseline: 1.3066x; 1.1258x over previous
import functools

import jax
import jax.numpy as jnp
from jax import lax
from jax.experimental import pallas as pl
from jax.experimental.pallas import tpu as pltpu
from jax.experimental.pallas import tpu_sc as plsc

NUM_EMBEDDINGS = 8192
EMBEDDING_DIM = 64
COMMITMENT_COST = 0.25

N_ROWS = 8 * 1024
ROW_BLK = 2048
CODE_BLK = 2048
N_RB = N_ROWS // ROW_BLK
N_CB = NUM_EMBEDDINGS // CODE_BLK


def _argmin_body(x_ref, e_ref, idx_ref, loss_ref, minv_ref, mini_ref,
                 mind_ref, acc_ref):
    i = pl.program_id(0)
    j = pl.program_id(1)

    x = x_ref[...]
    e = e_ref[...]
    xb = x.astype(jnp.bfloat16).astype(jnp.float32)
    mm = lax.dot_general(
        xb, e, (((1,), (1,)), ((), ())),
        preferred_element_type=jnp.float32)
    row_sq = jnp.sum(x * x, axis=1, keepdims=True)
    e_sq = jnp.sum(e * e, axis=1)[None, :]

    LN = 128
    NS = CODE_BLK // LN
    mv = None
    mi = None
    for s in range(NS):
        ds = ((row_sq + e_sq[:, s * LN:(s + 1) * LN])
              - 2.0 * mm[:, s * LN:(s + 1) * LN])
        if s == 0:
            mv = ds
            mi = jnp.zeros((ROW_BLK, LN), jnp.float32)
        else:
            lt = ds < mv
            mv = jnp.where(lt, ds, mv)
            mi = jnp.where(lt, jnp.float32(s), mi)

    lane = lax.broadcasted_iota(jnp.int32, (ROW_BLK, LN), 1).astype(jnp.float32)
    jf = mi * jnp.float32(LN) + lane
    bmin = jnp.min(mv, axis=1, keepdims=True)
    barg_f = jnp.min(jnp.where(mv == bmin, jf, jnp.float32(2 ** 30)),
                     axis=1, keepdims=True)
    barg = barg_f.astype(jnp.int32) + j * CODE_BLK

    bmin_r = bmin.astype(jnp.bfloat16).astype(jnp.float32)

    @pl.when(j == 0)
    def _():
        minv_ref[...] = bmin_r
        mini_ref[...] = barg
        mind_ref[...] = bmin

    @pl.when(j > 0)
    def _():
        better = bmin < minv_ref[...]
        minv_ref[...] = jnp.where(better, bmin_r, minv_ref[...])
        mini_ref[...] = jnp.where(better, barg, mini_ref[...])
        mind_ref[...] = jnp.where(better, bmin, mind_ref[...])

    @pl.when(j == N_CB - 1)
    def _():
        idx_ref[...] = mini_ref[...]
        part = jnp.sum(mind_ref[...])
        acc = jnp.where(i == 0, 0.0, acc_ref[0, 0]) + part
        acc_ref[0, 0] = acc
        scale = (1.0 + COMMITMENT_COST) / (N_ROWS * EMBEDDING_DIM)
        loss_ref[...] = jnp.full((1, 1), acc * scale, jnp.float32)


_argmin_call = pl.pallas_call(
    _argmin_body,
    grid=(N_RB, N_CB),
    in_specs=[
        pl.BlockSpec((ROW_BLK, EMBEDDING_DIM), lambda i, j: (i, 0)),
        pl.BlockSpec((CODE_BLK, EMBEDDING_DIM), lambda i, j: (j, 0)),
    ],
    out_specs=[
        pl.BlockSpec((ROW_BLK, 1), lambda i, j: (i, 0)),
        pl.BlockSpec((1, 1), lambda i, j: (0, 0)),
    ],
    out_shape=[
        jax.ShapeDtypeStruct((N_ROWS, 1), jnp.int32),
        jax.ShapeDtypeStruct((1, 1), jnp.float32),
    ],
    scratch_shapes=[
        pltpu.VMEM((ROW_BLK, 1), jnp.float32),
        pltpu.VMEM((ROW_BLK, 1), jnp.int32),
        pltpu.VMEM((ROW_BLK, 1), jnp.float32),
        pltpu.SMEM((1, 1), jnp.float32),
    ],
)


def _make_gather():
    nc, ns = 2, 16
    nw = nc * ns
    rows_per_w = N_ROWS // nw
    chunks = rows_per_w // 128

    mesh = plsc.VectorSubcoreMesh(core_axis_name="c", subcore_axis_name="s")

    @functools.partial(
        pl.kernel, mesh=mesh,
        compiler_params=pltpu.CompilerParams(use_tc_tiling_on_sc=False),
        out_type=jax.ShapeDtypeStruct((N_ROWS, EMBEDDING_DIM), jnp.float32),
        scratch_types=[
            pltpu.VMEM((chunks, 128), jnp.int32),
            pltpu.VMEM((rows_per_w, EMBEDDING_DIM), jnp.float32),
            pltpu.SemaphoreType.DMA,
        ],
    )
    def gather_k(table_hbm, idx_hbm, out_hbm, idx_v, rows_v, sem):
        wid = lax.axis_index("s") * nc + lax.axis_index("c")
        pltpu.sync_copy(idx_hbm.at[pl.ds(wid * chunks, chunks)], idx_v)
        cps = [
            pltpu.async_copy(table_hbm.at[idx_v.at[k]],
                             rows_v.at[pl.ds(k * 128, 128)], sem)
            for k in range(chunks)
        ]
        for cp in cps:
            cp.wait()
        pltpu.sync_copy(rows_v, out_hbm.at[pl.ds(wid * rows_per_w, rows_per_w)])

    return gather_k


_gather_cache = []


def _gather_call(embeddings, idx_rows):
    if not _gather_cache:
        _gather_cache.append(_make_gather())
    return _gather_cache[0](embeddings, idx_rows)


def kernel(inputs, embeddings):
    input_shape = inputs.shape
    flat = inputs.reshape(-1, EMBEDDING_DIM)
    idx2d, loss11 = _argmin_call(flat, embeddings)
    idx_rows = idx2d.reshape(-1, 128)
    quantized = _gather_call(embeddings, idx_rows).reshape(input_shape)
    return quantized, loss11[0, 0], idx2d
